# Initial kernel scaffold; baseline (speedup 1.0000x reference)
import jax
import jax.numpy as jnp
from jax.experimental import pallas as pl


def kernel(xs_al, xs_ac, xs_t, es0, es1, w0, w1, enc_al_weight, enc_ac_W, enc_ac_b, emb_test, mpnn_W, mpnn_b, decode_W, decode_b):
    raise NotImplementedError("write your pallas kernel here")


# speedup vs baseline: 1.2774x; 1.2774x over previous
import functools

import jax
import jax.numpy as jnp
from jax import lax
from jax.experimental import pallas as pl
from jax.experimental.pallas import tpu as pltpu
from jax.experimental.pallas import tpu_sc as plsc

N_A = 10000
N_T = 10000
E = 160000
D = 128
N_AL = 1000
DIM_AC = 32
N_LAYERS = 5
N_CLASSES = 3

NC = 2
NS = 16
NW = NC * NS
EPW = E // NW
CH = 128
NCH = 40
EPWP = NCH * CH
ACC_R = 10112
ZR = ACC_R // NS
OR = 624

GPW = 320
GCH = 64
GNCH = 5

_sc_mesh = plsc.VectorSubcoreMesh(core_axis_name="c", subcore_axis_name="s")


@functools.partial(
    pl.kernel,
    out_type=jax.ShapeDtypeStruct((NW * GPW, D), jnp.float32),
    mesh=_sc_mesh,
    scratch_types=[
        pltpu.VMEM((GNCH, GCH), jnp.int32),
        pltpu.VMEM((GPW, D), jnp.float32),
        pltpu.SemaphoreType.DMA,
    ],
)
def _sc_enc_gather(tab_hbm, idx_hbm, g_hbm, idx_v, rows_v, sem):
    cid = lax.axis_index("c")
    sid = lax.axis_index("s")
    wid = sid * NC + cid
    pltpu.sync_copy(idx_hbm.at[wid], idx_v)
    for i in range(GNCH):
        pltpu.async_copy(tab_hbm.at[idx_v.at[i]],
                         rows_v.at[pl.ds(i * GCH, GCH)], sem).wait()
    pltpu.sync_copy(rows_v, g_hbm.at[pl.ds(wid * GPW, GPW)])


@functools.partial(
    pl.kernel,
    out_type=(jax.ShapeDtypeStruct((NC, N_T, D), jnp.float32),
              jax.ShapeDtypeStruct((NC, N_A, D), jnp.float32)),
    mesh=_sc_mesh,
    scratch_types=[
        pltpu.VMEM((NCH, CH), jnp.int32),
        pltpu.VMEM((NCH, CH), jnp.int32),
        pltpu.VMEM((CH * 16,), jnp.float32),
        pltpu.VMEM((CH, D), jnp.float32),
        pltpu.MemorySpace.VMEM_SHARED((ACC_R, D), jnp.float32),
        pltpu.SemaphoreType.DMA,
    ],
)
def _sc_layer(y0, y1, si0, di0, w0p, si1, di1, w1p, z_hbm,
              out_t, out_a, sidx, didx, wv, rows, acc, sem):
    cid = lax.axis_index("c")
    sid = lax.axis_index("s")
    wid = sid * NC + cid

    def phase(y_hbm, si_hbm, di_hbm, w_hbm, out_hbm):
        pltpu.sync_copy(z_hbm, acc.at[pl.ds(sid * ZR, ZR)])
        pltpu.sync_copy(si_hbm.at[wid], sidx)
        pltpu.sync_copy(di_hbm.at[wid], didx)
        plsc.subcore_barrier()

        def chunk(j, carry):
            pltpu.sync_copy(w_hbm.at[wid, pl.ds(j * CH * 16, CH * 16)], wv)
            pltpu.async_copy(y_hbm.at[sidx.at[j]], rows, sem).wait()

            def scale(e, c2):
                wb = wv[pl.ds(e * 16, 16)]
                for k in range(D // 16):
                    rows[e, pl.ds(k * 16, 16)] = rows[e, pl.ds(k * 16, 16)] * wb
                return c2

            lax.fori_loop(0, CH, scale, 0, unroll=False)
            pltpu.sync_copy(rows, acc.at[didx.at[j]], add=True)
            return carry

        lax.fori_loop(0, NCH, chunk, 0, unroll=False)
        plsc.subcore_barrier()
        pltpu.sync_copy(acc.at[pl.ds(sid * OR, OR)],
                        out_hbm.at[cid, pl.ds(sid * OR, OR)])

        @pl.when(sid == NS - 1)
        def _():
            pltpu.sync_copy(acc.at[pl.ds(NS * OR, N_A - NS * OR)],
                            out_hbm.at[cid, pl.ds(NS * OR, N_A - NS * OR)])

        plsc.subcore_barrier()

    phase(y0, si0, di0, w0p, out_t)
    phase(y1, si1, di1, w1p, out_a)


RB = 1000
NBLK = N_A // RB

_full = lambda shape: pl.BlockSpec(shape, lambda i: tuple(0 for _ in shape))
_rows = lambda w: pl.BlockSpec((RB, w), lambda i: (i, 0))
_rows2 = lambda w: pl.BlockSpec((NC, RB, w), lambda i: (0, i, 0))


def _tc_encode_body(g, ac, wac, bac, emb, w00, b00, w01, b01, y0, y1):
    x0 = g[...] + jnp.dot(ac[...], wac[...],
                          preferred_element_type=jnp.float32) + bac[...]
    y0[...] = jnp.maximum(
        jnp.dot(x0, w00[...], preferred_element_type=jnp.float32) + b00[...], 0.0)
    y1row = jnp.maximum(
        jnp.dot(emb[...], w01[...], preferred_element_type=jnp.float32)
        + b01[...], 0.0)
    y1[...] = jnp.broadcast_to(y1row, (RB, D))


_tc_encode = pl.pallas_call(
    _tc_encode_body,
    grid=(NBLK,),
    in_specs=[_rows(D), _rows(DIM_AC), _full((DIM_AC, D)), _full((1, D)),
              _full((1, D)), _full((D, D)), _full((1, D)),
              _full((D, D)), _full((1, D))],
    out_specs=[_rows(D), _rows(D)],
    out_shape=[jax.ShapeDtypeStruct((N_A, D), jnp.float32),
               jax.ShapeDtypeStruct((N_T, D), jnp.float32)],
)


def _tc_combine_body(pt, pa, w0, b0, w1, b1, y0, y1):
    x0 = jnp.maximum(pa[0] + pa[1], 0.0)
    x1 = jnp.maximum(pt[0] + pt[1], 0.0)
    y0[...] = jnp.maximum(
        jnp.dot(x0, w0[...], preferred_element_type=jnp.float32) + b0[...], 0.0)
    y1[...] = jnp.maximum(
        jnp.dot(x1, w1[...], preferred_element_type=jnp.float32) + b1[...], 0.0)


_tc_combine = pl.pallas_call(
    _tc_combine_body,
    grid=(NBLK,),
    in_specs=[_rows2(D), _rows2(D), _full((D, D)), _full((1, D)),
              _full((D, D)), _full((1, D))],
    out_specs=[_rows(D), _rows(D)],
    out_shape=[jax.ShapeDtypeStruct((N_A, D), jnp.float32),
               jax.ShapeDtypeStruct((N_T, D), jnp.float32)],
)


def _tc_decode_body(pa, wp, bp, last, sm):
    x0 = jnp.maximum(pa[0] + pa[1], 0.0)
    l = jnp.dot(x0, wp[...], preferred_element_type=jnp.float32) + bp[...]
    m = jnp.max(l, axis=1, keepdims=True)
    ex = jnp.exp(l - m)
    last[...] = l
    sm[...] = ex / jnp.sum(ex, axis=1, keepdims=True)


_tc_decode = pl.pallas_call(
    _tc_decode_body,
    grid=(NBLK,),
    in_specs=[_rows2(D), _full((D, D)), _full((1, D))],
    out_specs=[_rows(D), _rows(D)],
    out_shape=[jax.ShapeDtypeStruct((N_A, D), jnp.float32),
               jax.ShapeDtypeStruct((N_A, D), jnp.float32)],
)


def _prep_edges(es, w):
    src = es[0].reshape(NW, EPW)
    dst = es[1].reshape(NW, EPW)
    wr = w.reshape(NW, EPW)
    pad = EPWP - EPW
    src_p = jnp.pad(src, ((0, 0), (0, pad)))
    dst_p = jnp.pad(dst, ((0, 0), (0, pad)), constant_values=N_A)
    w_p = jnp.pad(wr, ((0, 0), (0, pad)))
    w_b = jnp.broadcast_to(w_p[:, :, None], (NW, EPWP, 16)).reshape(NW, EPWP * 16)
    return (src_p.reshape(NW, NCH, CH), dst_p.reshape(NW, NCH, CH), w_b)


def kernel(xs_al, xs_ac, xs_t, es0, es1, w0, w1,
           enc_al_weight, enc_ac_W, enc_ac_b, emb_test,
           mpnn_W, mpnn_b, decode_W, decode_b):
    del xs_t

    si0, di0, w0p = _prep_edges(es0, w0)
    si1, di1, w1p = _prep_edges(es1, w1)
    z = jnp.zeros((ZR, D), jnp.float32)

    idx_p = jnp.pad(xs_al.astype(jnp.int32), (0, NW * GPW - N_A))
    g = _sc_enc_gather(enc_al_weight, idx_p.reshape(NW, GNCH, GCH))[:N_A]

    y0, y1 = _tc_encode(g, xs_ac, enc_ac_W, enc_ac_b.reshape(1, D),
                        emb_test, mpnn_W[0, 0], mpnn_b[0, 0].reshape(1, D),
                        mpnn_W[0, 1], mpnn_b[0, 1].reshape(1, D))

    for i in range(N_LAYERS):
        pt, pa = _sc_layer(y0, y1, si0, di0, w0p, si1, di1, w1p, z)
        if i + 1 < N_LAYERS:
            y0, y1 = _tc_combine(
                pt, pa, mpnn_W[i + 1, 0], mpnn_b[i + 1, 0].reshape(1, D),
                mpnn_W[i + 1, 1], mpnn_b[i + 1, 1].reshape(1, D))

    wp = jnp.zeros((D, D), jnp.float32).at[:, :N_CLASSES].set(decode_W)
    bp = jnp.full((1, D), -1e30, jnp.float32).at[0, :N_CLASSES].set(decode_b)
    last, sm = _tc_decode(pa, wp, bp)
    return (last[:, :N_CLASSES], sm[:, :N_CLASSES])


# speedup vs baseline: 1.6844x; 1.3186x over previous
import functools

import jax
import jax.numpy as jnp
from jax import lax
from jax.experimental import pallas as pl
from jax.experimental.pallas import tpu as pltpu
from jax.experimental.pallas import tpu_sc as plsc

N_A = 10000
N_T = 10000
E = 160000
D = 128
N_AL = 1000
DIM_AC = 32
N_LAYERS = 5
N_CLASSES = 3

NC = 2
NS = 16
NW = NC * NS
EPW = E // NW
CH = 64
NCH = 80
NBUF = 3
EPWP = NCH * CH
ACC_R = 10112
ZR = ACC_R // NS
OR = 624

GPW = 320
GCH = 64
GNCH = 5

_sc_mesh = plsc.VectorSubcoreMesh(core_axis_name="c", subcore_axis_name="s")


@functools.partial(
    pl.kernel,
    out_type=jax.ShapeDtypeStruct((NW * GPW, D), jnp.float32),
    mesh=_sc_mesh,
    scratch_types=[
        pltpu.VMEM((GNCH, GCH), jnp.int32),
        pltpu.VMEM((GCH, D), jnp.float32),
        pltpu.SemaphoreType.DMA,
    ],
)
def _sc_enc_gather(tab_hbm, idx_hbm, g_hbm, idx_v, rows_v, sem):
    cid = lax.axis_index("c")
    sid = lax.axis_index("s")
    wid = sid * NC + cid
    pltpu.sync_copy(idx_hbm.at[wid], idx_v)
    for i in range(GNCH):
        pltpu.async_copy(tab_hbm.at[idx_v.at[i]], rows_v, sem).wait()
        pltpu.sync_copy(rows_v, g_hbm.at[pl.ds(wid * GPW + i * GCH, GCH)])


@functools.partial(
    pl.kernel,
    out_type=(jax.ShapeDtypeStruct((NC, N_T, D), jnp.float32),
              jax.ShapeDtypeStruct((NC, N_A, D), jnp.float32)),
    mesh=_sc_mesh,
    scratch_types=[
        pltpu.VMEM((NCH, CH), jnp.int32),
        pltpu.VMEM((NCH, CH), jnp.int32),
        [pltpu.VMEM((CH * 16,), jnp.float32)] * NBUF,
        [pltpu.VMEM((CH, D), jnp.float32)] * NBUF,
        pltpu.MemorySpace.VMEM_SHARED((ACC_R, D), jnp.float32),
        [pltpu.SemaphoreType.DMA] * NBUF,
        [pltpu.SemaphoreType.DMA] * NBUF,
    ],
)
def _sc_layer(y0, y1, si0, di0, w0p, si1, di1, w1p, z_hbm,
              out_t, out_a, sidx, didx, wv, rows, acc, gsems, ssems):
    cid = lax.axis_index("c")
    sid = lax.axis_index("s")
    wid = sid * NC + cid
    CH16 = CH * 16

    def phase(y_hbm, si_hbm, di_hbm, w_hbm, out_hbm):
        pltpu.sync_copy(z_hbm, acc.at[pl.ds(sid * ZR, ZR)])
        pltpu.sync_copy(si_hbm.at[wid], sidx)
        pltpu.sync_copy(di_hbm.at[wid], didx)
        plsc.subcore_barrier()

        def fire_g(j, b):
            pltpu.async_copy(y_hbm.at[sidx.at[j]], rows[b], gsems[b])
            pltpu.async_copy(w_hbm.at[wid, pl.ds(j * CH16, CH16)],
                             wv[b], gsems[b])

        def wait_g(b):
            pltpu.make_async_copy(y_hbm.at[sidx.at[0]], rows[b],
                                  gsems[b]).wait()
            pltpu.make_async_copy(w_hbm.at[wid, pl.ds(0, CH16)], wv[b],
                                  gsems[b]).wait()

        def fire_s(j, b):
            pltpu.async_copy(rows[b], acc.at[didx.at[j]], ssems[b],
                             add=True)

        def wait_s(b):
            pltpu.make_async_copy(rows[b], acc.at[didx.at[0]],
                                  ssems[b]).wait()

        def scale(b):
            rb = rows[b]
            wb_ref = wv[b]

            def body(e, c2):
                wb = wb_ref[pl.ds(e * 16, 16)]
                for k in range(D // 16):
                    rb[e, pl.ds(k * 16, 16)] = rb[e, pl.ds(k * 16, 16)] * wb
                return c2

            lax.fori_loop(0, CH, body, 0, unroll=4)

        def step(j, b, wait_prev, fire_next):
            wait_g(b)
            scale(b)
            fire_s(j, b)
            b2 = (b + 2) % NBUF
            if wait_prev:
                wait_s(b2)
            if fire_next:
                fire_g(j + 2, b2)

        fire_g(0, 0)
        fire_g(1, 1)
        step(0, 0, False, True)
        step(1, 1, True, True)
        step(2, 2, True, True)

        def steady(j3, carry):
            for b in range(NBUF):
                step(j3 * NBUF + b, b, True, True)
            return carry

        lax.fori_loop(1, NCH // NBUF, steady, 0, unroll=False)
        step(NCH - 2, (NCH - 2) % NBUF, True, False)
        step(NCH - 1, (NCH - 1) % NBUF, True, False)
        wait_s((NCH - 1) % NBUF)
        plsc.subcore_barrier()
        pltpu.sync_copy(acc.at[pl.ds(sid * OR, OR)],
                        out_hbm.at[cid, pl.ds(sid * OR, OR)])

        @pl.when(sid == NS - 1)
        def _():
            pltpu.sync_copy(acc.at[pl.ds(NS * OR, N_A - NS * OR)],
                            out_hbm.at[cid, pl.ds(NS * OR, N_A - NS * OR)])

        plsc.subcore_barrier()

    phase(y0, si0, di0, w0p, out_t)
    phase(y1, si1, di1, w1p, out_a)


RB = 1000
NBLK = N_A // RB

_full = lambda shape: pl.BlockSpec(shape, lambda i: tuple(0 for _ in shape))
_rows = lambda w: pl.BlockSpec((RB, w), lambda i: (i, 0))
_rows2 = lambda w: pl.BlockSpec((NC, RB, w), lambda i: (0, i, 0))


def _tc_encode_body(g, ac, wac, bac, emb, w00, b00, w01, b01, y0, y1):
    x0 = g[...] + jnp.dot(ac[...], wac[...],
                          preferred_element_type=jnp.float32) + bac[...]
    y0[...] = jnp.maximum(
        jnp.dot(x0, w00[...], preferred_element_type=jnp.float32) + b00[...], 0.0)
    y1row = jnp.maximum(
        jnp.dot(emb[...], w01[...], preferred_element_type=jnp.float32)
        + b01[...], 0.0)
    y1[...] = jnp.broadcast_to(y1row, (RB, D))


_tc_encode = pl.pallas_call(
    _tc_encode_body,
    grid=(NBLK,),
    in_specs=[_rows(D), _rows(DIM_AC), _full((DIM_AC, D)), _full((1, D)),
              _full((1, D)), _full((D, D)), _full((1, D)),
              _full((D, D)), _full((1, D))],
    out_specs=[_rows(D), _rows(D)],
    out_shape=[jax.ShapeDtypeStruct((N_A, D), jnp.float32),
               jax.ShapeDtypeStruct((N_T, D), jnp.float32)],
)


def _tc_combine_body(pt, pa, w0, b0, w1, b1, y0, y1):
    x0 = jnp.maximum(pa[0] + pa[1], 0.0)
    x1 = jnp.maximum(pt[0] + pt[1], 0.0)
    y0[...] = jnp.maximum(
        jnp.dot(x0, w0[...], preferred_element_type=jnp.float32) + b0[...], 0.0)
    y1[...] = jnp.maximum(
        jnp.dot(x1, w1[...], preferred_element_type=jnp.float32) + b1[...], 0.0)


_tc_combine = pl.pallas_call(
    _tc_combine_body,
    grid=(NBLK,),
    in_specs=[_rows2(D), _rows2(D), _full((D, D)), _full((1, D)),
              _full((D, D)), _full((1, D))],
    out_specs=[_rows(D), _rows(D)],
    out_shape=[jax.ShapeDtypeStruct((N_A, D), jnp.float32),
               jax.ShapeDtypeStruct((N_T, D), jnp.float32)],
)


def _tc_decode_body(pa, wp, bp, last, sm):
    x0 = jnp.maximum(pa[0] + pa[1], 0.0)
    l = jnp.dot(x0, wp[...], preferred_element_type=jnp.float32) + bp[...]
    m = jnp.max(l, axis=1, keepdims=True)
    ex = jnp.exp(l - m)
    last[...] = l
    sm[...] = ex / jnp.sum(ex, axis=1, keepdims=True)


_tc_decode = pl.pallas_call(
    _tc_decode_body,
    grid=(NBLK,),
    in_specs=[_rows2(D), _full((D, D)), _full((1, D))],
    out_specs=[_rows(D), _rows(D)],
    out_shape=[jax.ShapeDtypeStruct((N_A, D), jnp.float32),
               jax.ShapeDtypeStruct((N_A, D), jnp.float32)],
)


def _prep_edges(es, w):
    src = es[0].reshape(NW, EPW)
    dst = es[1].reshape(NW, EPW)
    wr = w.reshape(NW, EPW)
    pad = EPWP - EPW
    src_p = jnp.pad(src, ((0, 0), (0, pad)))
    dst_p = jnp.pad(dst, ((0, 0), (0, pad)), constant_values=N_A)
    w_p = jnp.pad(wr, ((0, 0), (0, pad)))
    w_b = jnp.broadcast_to(w_p[:, :, None], (NW, EPWP, 16)).reshape(NW, EPWP * 16)
    return (src_p.reshape(NW, NCH, CH), dst_p.reshape(NW, NCH, CH), w_b)


def kernel(xs_al, xs_ac, xs_t, es0, es1, w0, w1,
           enc_al_weight, enc_ac_W, enc_ac_b, emb_test,
           mpnn_W, mpnn_b, decode_W, decode_b):
    del xs_t

    si0, di0, w0p = _prep_edges(es0, w0)
    si1, di1, w1p = _prep_edges(es1, w1)
    z = jnp.zeros((ZR, D), jnp.float32)

    idx_p = jnp.pad(xs_al.astype(jnp.int32), (0, NW * GPW - N_A))
    g = _sc_enc_gather(enc_al_weight, idx_p.reshape(NW, GNCH, GCH))[:N_A]

    y0, y1 = _tc_encode(g, xs_ac, enc_ac_W, enc_ac_b.reshape(1, D),
                        emb_test, mpnn_W[0, 0], mpnn_b[0, 0].reshape(1, D),
                        mpnn_W[0, 1], mpnn_b[0, 1].reshape(1, D))

    for i in range(N_LAYERS):
        pt, pa = _sc_layer(y0, y1, si0, di0, w0p, si1, di1, w1p, z)
        if i + 1 < N_LAYERS:
            y0, y1 = _tc_combine(
                pt, pa, mpnn_W[i + 1, 0], mpnn_b[i + 1, 0].reshape(1, D),
                mpnn_W[i + 1, 1], mpnn_b[i + 1, 1].reshape(1, D))

    wp = jnp.zeros((D, D), jnp.float32).at[:, :N_CLASSES].set(decode_W)
    bp = jnp.full((1, D), -1e30, jnp.float32).at[0, :N_CLASSES].set(decode_b)
    last, sm = _tc_decode(pa, wp, bp)
    return (last[:, :N_CLASSES], sm[:, :N_CLASSES])
